# Initial kernel scaffold; baseline (speedup 1.0000x reference)
import functools

import jax
import jax.numpy as jnp
from jax import lax
from jax.experimental import pallas as pl
from jax.experimental.pallas import tpu as pltpu
from jax.experimental.pallas import tpu_sc as plsc

N = 10000
E = 320000
D = 128

NC = 2
NS = 16
NW = NC * NS

N_PAD = 10240
ROWS_PER_TILE = N_PAD // NS
E_PAD = 327680
EROWS = E_PAD // 128
EROWS_PER_TILE = EROWS // NW
CHUNK_ROWS = 4
NCHUNK = EROWS_PER_TILE // CHUNK_ROWS

_mesh = plsc.VectorSubcoreMesh(core_axis_name="c", subcore_axis_name="s")


def _fill_rows(ref, nrows, ncols, value):
    @pl.loop(0, nrows)
    def _(i):
        for j in range(ncols // 16):
            ref[i, pl.ds(j * 16, 16)] = jnp.full((16,), value, jnp.float32)


@functools.partial(
    pl.kernel,
    out_type=jax.ShapeDtypeStruct((NC, N_PAD, 16), jnp.float32),
    mesh=_mesh,
    scratch_types=[
        pltpu.VMEM_SHARED((N_PAD, 16), jnp.float32),
        pltpu.VMEM((ROWS_PER_TILE, 16), jnp.float32),
        pltpu.VMEM((128, 16), jnp.float32),
        pltpu.VMEM((CHUNK_ROWS, 128), jnp.int32),
    ],
)
def _deg_kernel(dst_hbm, out_hbm, acc, zbuf, ones_v, didx):
    cid = lax.axis_index("c")
    sid = lax.axis_index("s")
    wid = sid * NC + cid

    _fill_rows(zbuf, ROWS_PER_TILE, 16, 0.0)
    _fill_rows(ones_v, 128, 16, 1.0)
    pltpu.sync_copy(zbuf, acc.at[pl.ds(sid * ROWS_PER_TILE, ROWS_PER_TILE)])
    plsc.subcore_barrier()

    ebase = wid * EROWS_PER_TILE

    @pl.loop(0, NCHUNK)
    def _(k):
        pltpu.sync_copy(dst_hbm.at[pl.ds(ebase + k * CHUNK_ROWS, CHUNK_ROWS)], didx)
        for j in range(CHUNK_ROWS):
            pltpu.sync_copy(ones_v, acc.at[didx.at[j]], add=True)

    plsc.subcore_barrier()
    pltpu.sync_copy(
        acc.at[pl.ds(sid * ROWS_PER_TILE, ROWS_PER_TILE)],
        out_hbm.at[cid, pl.ds(sid * ROWS_PER_TILE, ROWS_PER_TILE)],
    )


@functools.partial(
    pl.kernel,
    out_type=jax.ShapeDtypeStruct((NC, N_PAD, D), jnp.float32),
    mesh=_mesh,
    scratch_types=[
        pltpu.VMEM_SHARED((N_PAD, D), jnp.float32),
        pltpu.VMEM((CHUNK_ROWS * 128, D), jnp.float32),
        pltpu.VMEM((CHUNK_ROWS, 128), jnp.int32),
        pltpu.VMEM((CHUNK_ROWS, 128), jnp.int32),
        pltpu.SemaphoreType.DMA,
    ],
)
def _agg_kernel(u_hbm, src_hbm, dst_hbm, out_hbm, acc, rows, sidx, didx, sem):
    cid = lax.axis_index("c")
    sid = lax.axis_index("s")
    wid = sid * NC + cid

    _fill_rows(rows, CHUNK_ROWS * 128, D, 0.0)
    base = sid * ROWS_PER_TILE
    pltpu.sync_copy(rows, acc.at[pl.ds(base, CHUNK_ROWS * 128)])
    pltpu.sync_copy(
        rows.at[pl.ds(0, ROWS_PER_TILE - CHUNK_ROWS * 128)],
        acc.at[pl.ds(base + CHUNK_ROWS * 128, ROWS_PER_TILE - CHUNK_ROWS * 128)],
    )
    plsc.subcore_barrier()

    ebase = wid * EROWS_PER_TILE

    @pl.loop(0, NCHUNK)
    def _(k):
        pltpu.sync_copy(src_hbm.at[pl.ds(ebase + k * CHUNK_ROWS, CHUNK_ROWS)], sidx)
        pltpu.sync_copy(dst_hbm.at[pl.ds(ebase + k * CHUNK_ROWS, CHUNK_ROWS)], didx)
        for j in range(CHUNK_ROWS):
            pltpu.async_copy(
                u_hbm.at[sidx.at[j]], rows.at[pl.ds(j * 128, 128)], sem
            ).wait()
            pltpu.sync_copy(rows.at[pl.ds(j * 128, 128)], acc.at[didx.at[j]], add=True)

    plsc.subcore_barrier()
    pltpu.sync_copy(
        acc.at[pl.ds(sid * ROWS_PER_TILE, ROWS_PER_TILE)],
        out_hbm.at[cid, pl.ds(sid * ROWS_PER_TILE, ROWS_PER_TILE)],
    )


_BLK = 1024
_GRID = N_PAD // _BLK

_row_spec = pl.BlockSpec((_BLK, D), lambda i: (i, 0))
_col_spec = pl.BlockSpec((_BLK, 1), lambda i: (i, 0))
_mat_spec = pl.BlockSpec((D, D), lambda i: (0, 0))
_bias_spec = pl.BlockSpec((1, D), lambda i: (0, 0))


def _proj_body(x_ref, w_ref, dp0_ref, dp1_ref, u_ref, dinv_ref):
    deg = jnp.maximum(dp0_ref[...] + dp1_ref[...], 1.0)
    dv = lax.rsqrt(deg)
    dinv_ref[...] = dv
    h = jnp.dot(x_ref[...], w_ref[...], preferred_element_type=jnp.float32)
    u_ref[...] = h * dv


_proj = pl.pallas_call(
    _proj_body,
    grid=(_GRID,),
    in_specs=[_row_spec, _mat_spec, _col_spec, _col_spec],
    out_specs=[_row_spec, _col_spec],
    out_shape=[
        jax.ShapeDtypeStruct((N_PAD, D), jnp.float32),
        jax.ShapeDtypeStruct((N_PAD, 1), jnp.float32),
    ],
)


def _mid_body(a0_ref, a1_ref, u_ref, dv_ref, b_ref, w_ref, out_ref):
    dv = dv_ref[...]
    t = (a0_ref[...] + a1_ref[...] + u_ref[...]) * dv + b_ref[...]
    z = jnp.maximum(t, 0.0)
    out_ref[...] = jnp.dot(z, w_ref[...], preferred_element_type=jnp.float32) * dv


_mid = pl.pallas_call(
    _mid_body,
    grid=(_GRID,),
    in_specs=[_row_spec, _row_spec, _row_spec, _col_spec, _bias_spec, _mat_spec],
    out_specs=_row_spec,
    out_shape=jax.ShapeDtypeStruct((N_PAD, D), jnp.float32),
)


def _final_body(a0_ref, a1_ref, u_ref, dv_ref, b_ref, out_ref):
    t = (a0_ref[...] + a1_ref[...] + u_ref[...]) * dv_ref[...] + b_ref[...]
    out_ref[...] = jnp.maximum(t, 0.0)


_final = pl.pallas_call(
    _final_body,
    grid=(_GRID,),
    in_specs=[_row_spec, _row_spec, _row_spec, _col_spec, _bias_spec],
    out_specs=_row_spec,
    out_shape=jax.ShapeDtypeStruct((N_PAD, D), jnp.float32),
)


def kernel(x, edge_index, W0, b0, W1, b1):
    ei = jnp.concatenate(
        [edge_index, jnp.full((2, E_PAD - E), N, edge_index.dtype)], axis=1
    ).astype(jnp.int32)
    src2d = ei[0].reshape(EROWS, 128)
    dst2d = ei[1].reshape(EROWS, 128)
    x_pad = jnp.pad(x, ((0, N_PAD - N), (0, 0)))

    deg16 = _deg_kernel(dst2d)
    dp0 = deg16[0, :, 0:1] + 1.0
    dp1 = deg16[1, :, 0:1]

    u0, dinv = _proj(x_pad, W0, dp0, dp1)
    agg0 = _agg_kernel(u0, src2d, dst2d)
    u1 = _mid(agg0[0], agg0[1], u0, dinv, b0.reshape(1, D), W1)
    agg1 = _agg_kernel(u1, src2d, dst2d)
    out = _final(agg1[0], agg1[1], u1, dinv, b1.reshape(1, D))
    return out[:N]


# speedup vs baseline: 7.0599x; 7.0599x over previous
import functools

import jax
import jax.numpy as jnp
from jax import lax
from jax.experimental import pallas as pl
from jax.experimental.pallas import tpu as pltpu
from jax.experimental.pallas import tpu_sc as plsc

N = 10000
E = 320000
D = 128

NC = 2
NS = 16
NW = NC * NS

N_PAD = 10240
ROWS_PER_TILE = N_PAD // NS
E_PAD = 327680
EROWS = E_PAD // 128
EROWS_PER_TILE = EROWS // NW
CHUNK_ROWS = 2
NCHUNK = EROWS_PER_TILE // CHUNK_ROWS

_mesh = plsc.VectorSubcoreMesh(core_axis_name="c", subcore_axis_name="s")


def _fill_rows(ref, nrows, ncols, value):
    @pl.loop(0, nrows)
    def _(i):
        for j in range(ncols // 16):
            ref[i, pl.ds(j * 16, 16)] = jnp.full((16,), value, jnp.float32)


@functools.partial(
    pl.kernel,
    out_type=jax.ShapeDtypeStruct((NC, N_PAD, D), jnp.float32),
    mesh=_mesh,
    scratch_types=[
        pltpu.VMEM_SHARED((N_PAD, D), jnp.float32),
        pltpu.VMEM((CHUNK_ROWS * 128, D), jnp.float32),
        pltpu.VMEM((CHUNK_ROWS, 128), jnp.int32),
    ],
)
def _deg_kernel(dst_hbm, out_hbm, acc, rows, didx):
    cid = lax.axis_index("c")
    sid = lax.axis_index("s")
    wid = sid * NC + cid

    _fill_rows(rows, CHUNK_ROWS * 128, D, 0.0)
    base = sid * ROWS_PER_TILE
    zrows = CHUNK_ROWS * 128
    for t in range(ROWS_PER_TILE // zrows):
        pltpu.sync_copy(rows, acc.at[pl.ds(base + t * zrows, zrows)])
    rem = ROWS_PER_TILE % zrows
    if rem:
        pltpu.sync_copy(
            rows.at[pl.ds(0, rem)],
            acc.at[pl.ds(base + (ROWS_PER_TILE // zrows) * zrows, rem)],
        )
    _fill_rows(rows, 128, D, 1.0)
    plsc.subcore_barrier()

    ebase = wid * EROWS_PER_TILE

    @pl.loop(0, NCHUNK)
    def _(k):
        pltpu.sync_copy(dst_hbm.at[pl.ds(ebase + k * CHUNK_ROWS, CHUNK_ROWS)], didx)
        for j in range(CHUNK_ROWS):
            pltpu.sync_copy(rows.at[pl.ds(0, 128)], acc.at[didx.at[j]], add=True)

    plsc.subcore_barrier()
    pltpu.sync_copy(
        acc.at[pl.ds(sid * ROWS_PER_TILE, ROWS_PER_TILE)],
        out_hbm.at[cid, pl.ds(sid * ROWS_PER_TILE, ROWS_PER_TILE)],
    )


@functools.partial(
    pl.kernel,
    out_type=jax.ShapeDtypeStruct((NC, N_PAD, D), jnp.float32),
    mesh=_mesh,
    scratch_types=[
        pltpu.VMEM_SHARED((N_PAD, D), jnp.float32),
        pltpu.VMEM((CHUNK_ROWS * 128, D), jnp.float32),
        pltpu.VMEM((CHUNK_ROWS, 128), jnp.int32),
        pltpu.VMEM((CHUNK_ROWS, 128), jnp.int32),
        pltpu.SemaphoreType.DMA,
    ],
)
def _agg_kernel(u_hbm, src_hbm, dst_hbm, out_hbm, acc, rows, sidx, didx, sem):
    cid = lax.axis_index("c")
    sid = lax.axis_index("s")
    wid = sid * NC + cid

    _fill_rows(rows, CHUNK_ROWS * 128, D, 0.0)
    base = sid * ROWS_PER_TILE
    zrows = CHUNK_ROWS * 128
    for t in range(ROWS_PER_TILE // zrows):
        pltpu.sync_copy(rows, acc.at[pl.ds(base + t * zrows, zrows)])
    rem = ROWS_PER_TILE % zrows
    if rem:
        pltpu.sync_copy(
            rows.at[pl.ds(0, rem)],
            acc.at[pl.ds(base + (ROWS_PER_TILE // zrows) * zrows, rem)],
        )
    plsc.subcore_barrier()

    ebase = wid * EROWS_PER_TILE

    @pl.loop(0, NCHUNK)
    def _(k):
        pltpu.sync_copy(src_hbm.at[pl.ds(ebase + k * CHUNK_ROWS, CHUNK_ROWS)], sidx)
        pltpu.sync_copy(dst_hbm.at[pl.ds(ebase + k * CHUNK_ROWS, CHUNK_ROWS)], didx)
        for j in range(CHUNK_ROWS):
            pltpu.async_copy(
                u_hbm.at[sidx.at[j]], rows.at[pl.ds(j * 128, 128)], sem
            ).wait()
            pltpu.sync_copy(rows.at[pl.ds(j * 128, 128)], acc.at[didx.at[j]], add=True)

    plsc.subcore_barrier()
    pltpu.sync_copy(
        acc.at[pl.ds(sid * ROWS_PER_TILE, ROWS_PER_TILE)],
        out_hbm.at[cid, pl.ds(sid * ROWS_PER_TILE, ROWS_PER_TILE)],
    )


_BLK = 1024
_GRID = N_PAD // _BLK

_row_spec = pl.BlockSpec((_BLK, D), lambda i: (i, 0))
_col_spec = pl.BlockSpec((_BLK, 1), lambda i: (i, 0))
_mat_spec = pl.BlockSpec((D, D), lambda i: (0, 0))
_bias_spec = pl.BlockSpec((1, D), lambda i: (0, 0))


def _proj_body(x_ref, w_ref, dp0_ref, dp1_ref, u_ref, dinv_ref):
    deg = jnp.maximum(dp0_ref[...] + dp1_ref[...] + 1.0, 1.0)
    dv = lax.rsqrt(deg)
    dinv_ref[...] = dv
    h = jnp.dot(x_ref[...], w_ref[...], preferred_element_type=jnp.float32)
    u_ref[...] = h * dv


_proj = pl.pallas_call(
    _proj_body,
    grid=(_GRID,),
    in_specs=[_row_spec, _mat_spec, _col_spec, _col_spec],
    out_specs=[_row_spec, _col_spec],
    out_shape=[
        jax.ShapeDtypeStruct((N_PAD, D), jnp.float32),
        jax.ShapeDtypeStruct((N_PAD, 1), jnp.float32),
    ],
)


def _mid_body(a0_ref, a1_ref, u_ref, dv_ref, b_ref, w_ref, out_ref):
    dv = dv_ref[...]
    t = (a0_ref[...] + a1_ref[...] + u_ref[...]) * dv + b_ref[...]
    z = jnp.maximum(t, 0.0)
    out_ref[...] = jnp.dot(z, w_ref[...], preferred_element_type=jnp.float32) * dv


_mid = pl.pallas_call(
    _mid_body,
    grid=(_GRID,),
    in_specs=[_row_spec, _row_spec, _row_spec, _col_spec, _bias_spec, _mat_spec],
    out_specs=_row_spec,
    out_shape=jax.ShapeDtypeStruct((N_PAD, D), jnp.float32),
)


def _final_body(a0_ref, a1_ref, u_ref, dv_ref, b_ref, out_ref):
    t = (a0_ref[...] + a1_ref[...] + u_ref[...]) * dv_ref[...] + b_ref[...]
    out_ref[...] = jnp.maximum(t, 0.0)


_final = pl.pallas_call(
    _final_body,
    grid=(_GRID,),
    in_specs=[_row_spec, _row_spec, _row_spec, _col_spec, _bias_spec],
    out_specs=_row_spec,
    out_shape=jax.ShapeDtypeStruct((N_PAD, D), jnp.float32),
)


def kernel(x, edge_index, W0, b0, W1, b1):
    ei = jnp.concatenate(
        [edge_index, jnp.full((2, E_PAD - E), N, edge_index.dtype)], axis=1
    ).astype(jnp.int32)
    src2d = ei[0].reshape(EROWS, 128)
    dst2d = ei[1].reshape(EROWS, 128)
    x_pad = jnp.pad(x, ((0, N_PAD - N), (0, 0)))

    deg16 = _deg_kernel(dst2d)
    dp0 = deg16[0, :, 0:1]
    dp1 = deg16[1, :, 0:1]

    u0, dinv = _proj(x_pad, W0, dp0, dp1)
    agg0 = _agg_kernel(u0, src2d, dst2d)
    u1 = _mid(agg0[0], agg0[1], u0, dinv, b0.reshape(1, D), W1)
    agg1 = _agg_kernel(u1, src2d, dst2d)
    out = _final(agg1[0], agg1[1], u1, dinv, b1.reshape(1, D))
    return out[:N]
